# baseline (speedup 1.0000x reference)
import functools

import jax
import jax.numpy as jnp
from jax import lax
from jax.experimental import pallas as pl
from jax.experimental.pallas import tpu as pltpu
from jax.experimental.pallas import tpu_sc as plsc

_REL_DIM = 64
_ROW = 128
_K = 8


def _build(B_total):
    NW = 32
    b_per_w = B_total // NW
    C = _K * _ROW
    nchunk = b_per_w // C
    nblk = B_total // _ROW
    blk_per_w = b_per_w // _ROW

    mesh = plsc.VectorSubcoreMesh(core_axis_name="c", subcore_axis_name="s")

    @functools.partial(
        pl.kernel,
        mesh=mesh,
        out_type=jax.ShapeDtypeStruct((nblk, _ROW, _REL_DIM), jnp.float32),
        scratch_types=[
            pltpu.VMEM((_K, _ROW), jnp.int32),
            pltpu.VMEM((_K, _ROW, _REL_DIM), jnp.float32),
            pltpu.SemaphoreType.DMA,
        ],
        compiler_params=pltpu.CompilerParams(use_tc_tiling_on_sc=False),
    )
    def k(ctx_hbm, table_hbm, out_hbm, idx_v, rows_v, sem):
        wid = lax.axis_index("s") * 2 + lax.axis_index("c")
        blk0 = wid * blk_per_w

        def body(j, carry):
            b = blk0 + j * _K
            pltpu.sync_copy(ctx_hbm.at[pl.ds(b, _K)], idx_v)
            for t in range(_K):
                pltpu.async_copy(table_hbm.at[idx_v.at[t]], rows_v.at[t], sem)
            for t in range(_K):
                pltpu.make_async_copy(
                    table_hbm.at[idx_v.at[t]], rows_v.at[t], sem
                ).wait()
            pltpu.sync_copy(rows_v, out_hbm.at[pl.ds(b, _K)])
            return carry

        lax.fori_loop(0, nchunk, body, 0)

    return k


def kernel(context, table):
    B, H = context.shape
    B_total = B * H
    ctx2d = context.reshape(B_total // _ROW, _ROW)
    out = _build(B_total)(ctx2d, table)
    return out.reshape(B, H, _REL_DIM)


# speedup vs baseline: 1.0299x; 1.0299x over previous
import functools

import jax
import jax.numpy as jnp
from jax import lax
from jax.experimental import pallas as pl
from jax.experimental.pallas import tpu as pltpu
from jax.experimental.pallas import tpu_sc as plsc

_REL_DIM = 64
_ROW = 128
_K = 5
_NBUF = 2


def _build(B_total):
    NW = 32
    b_per_w = B_total // NW
    C = _K * _ROW
    nchunk = b_per_w // C
    nblk = B_total // _ROW
    blk_per_w = b_per_w // _ROW
    assert nchunk % _NBUF == 0

    mesh = plsc.VectorSubcoreMesh(core_axis_name="c", subcore_axis_name="s")

    @functools.partial(
        pl.kernel,
        mesh=mesh,
        out_type=jax.ShapeDtypeStruct((nblk, _ROW, _REL_DIM), jnp.float32),
        scratch_types=[
            pltpu.VMEM((_NBUF, _K, _ROW), jnp.int32),
            pltpu.VMEM((_NBUF, _K, _ROW, _REL_DIM), jnp.float32),
            pltpu.SemaphoreType.DMA((_NBUF,)),
            pltpu.SemaphoreType.DMA((_NBUF,)),
            pltpu.SemaphoreType.DMA((_NBUF,)),
        ],
        compiler_params=pltpu.CompilerParams(use_tc_tiling_on_sc=False),
    )
    def k(ctx_hbm, table_hbm, out_hbm, idx_v, rows_v, sem_i, sem_g, sem_o):
        wid = lax.axis_index("s") * 2 + lax.axis_index("c")
        blk0 = wid * blk_per_w

        def fire(j, b):
            for t in range(_K):
                pltpu.async_copy(
                    table_hbm.at[idx_v.at[b, t]], rows_v.at[b, t], sem_g.at[b]
                )

        def drain(b):
            for t in range(_K):
                pltpu.make_async_copy(
                    table_hbm.at[idx_v.at[b, t]], rows_v.at[b, t], sem_g.at[b]
                ).wait()

        for b in range(_NBUF):
            pltpu.sync_copy(ctx_hbm.at[pl.ds(blk0 + b * _K, _K)], idx_v.at[b])
            fire(b, b)

        def group(i, carry):
            for b in range(_NBUF):
                j = i * _NBUF + b
                blk = blk0 + j * _K
                drain(b)

                @pl.when(j + _NBUF < nchunk)
                def _prefetch_idx():
                    pltpu.async_copy(
                        ctx_hbm.at[pl.ds(blk + _NBUF * _K, _K)],
                        idx_v.at[b],
                        sem_i.at[b],
                    )

                pltpu.async_copy(rows_v.at[b], out_hbm.at[pl.ds(blk, _K)], sem_o.at[b])

                @pl.when(j + _NBUF < nchunk)
                def _next_gather():
                    pltpu.make_async_copy(
                        rows_v.at[b], out_hbm.at[pl.ds(blk, _K)], sem_o.at[b]
                    ).wait()
                    pltpu.make_async_copy(
                        ctx_hbm.at[pl.ds(blk + _NBUF * _K, _K)],
                        idx_v.at[b],
                        sem_i.at[b],
                    ).wait()
                    fire(j + _NBUF, b)

            return carry

        lax.fori_loop(0, nchunk // _NBUF, group, 0)

        for b in range(_NBUF):
            blk = blk0 + (nchunk - _NBUF + b) * _K
            pltpu.make_async_copy(
                rows_v.at[b], out_hbm.at[pl.ds(blk, _K)], sem_o.at[b]
            ).wait()

    return k


def kernel(context, table):
    B, H = context.shape
    B_total = B * H
    ctx2d = context.reshape(B_total // _ROW, _ROW)
    out = _build(B_total)(ctx2d, table)
    return out.reshape(B, H, _REL_DIM)
